# baseline (speedup 1.0000x reference)
import jax
import jax.numpy as jnp
from jax.experimental import pallas as pl
from jax.experimental.pallas import tpu as pltpu


def _embed_add_kernel(months_ref, x_ref, ce_ref, pe_ref, mt_ref, o_ref):
    t = pe_ref.shape[0]
    n = ce_ref.shape[-1]
    x = x_ref[...]
    ce = ce_ref[...]
    pe = pe_ref[...]
    me = jnp.stack([mt_ref[months_ref[0, tt], :] for tt in range(t)])
    o_ref[..., 0:n] = x[..., 0:n] + ce[None, None, None, :, :]
    o_ref[..., n:2 * n] = x[..., n:2 * n] + pe[None, None, :, None, :]
    o_ref[..., 2 * n:3 * n] = x[..., 2 * n:3 * n] + me[None, None, :, None, :]
    o_ref[..., 3 * n:] = x[..., 3 * n:]


def _one_batch(x, bi, months_b, channel_embed, pos8, month_table):
    b, hw, t, b_s, d = x.shape
    n = d // 4
    br = 32
    return pl.pallas_call(
        _embed_add_kernel,
        grid_spec=pltpu.PrefetchScalarGridSpec(
            num_scalar_prefetch=1,
            grid=(hw // br,),
            in_specs=[
                pl.BlockSpec((1, br, t, b_s, d), lambda j, m_ref: (bi, j, 0, 0, 0)),
                pl.BlockSpec((b_s, n), lambda j, m_ref: (0, 0)),
                pl.BlockSpec((t, n), lambda j, m_ref: (0, 0)),
                pl.BlockSpec(month_table.shape, lambda j, m_ref: (0, 0)),
            ],
            out_specs=pl.BlockSpec((1, br, t, b_s, d), lambda j, m_ref: (0, j, 0, 0, 0)),
        ),
        out_shape=jax.ShapeDtypeStruct((1, hw, t, b_s, d), x.dtype),
        compiler_params=pltpu.CompilerParams(
            dimension_semantics=("arbitrary",),
        ),
    )(months_b, x, channel_embed, pos8, month_table)


def kernel(sensor_tokens, timestamps, channel_embed, pos_embed, month_table):
    b, h, w, t, b_s, d = sensor_tokens.shape
    hw = h * w
    x = sensor_tokens.reshape(b, hw, t, b_s, d)
    months = timestamps[:, :, 1].astype(jnp.int32)
    pos8 = pos_embed[:t]
    outs = [
        _one_batch(x, i, months[i:i + 1], channel_embed, pos8, month_table)
        for i in range(b)
    ]
    out = jnp.concatenate(outs, axis=0)
    return out.reshape(b, h, w, t, b_s, d)
